# baseline (speedup 1.0000x reference)
import jax
import jax.numpy as jnp
from jax.experimental import pallas as pl
from jax.experimental.pallas import tpu as pltpu

B, L, D = 8, 2048, 128
TIME_MEAN, TIME_STD = 43.8756927994, 51.4811932987
DIST_MEAN, DIST_STD = 0.274716042312, 0.127051674693


def _shift_down(a):
    r = pltpu.roll(a, shift=1, axis=0)
    row = jax.lax.broadcasted_iota(jnp.int32, a.shape, 0)
    return jnp.where(row == 0, 0.0, r)


def _shift_up(a):
    r = pltpu.roll(a, shift=a.shape[0] - 1, axis=0)
    row = jax.lax.broadcasted_iota(jnp.int32, a.shape, 0)
    return jnp.where(row == a.shape[0] - 1, 0.0, r)


def _body(dis_ref, x_ref, wp1_ref, bp1_ref, ws1_ref, wn1_ref, b1_ref,
          wp3_ref, bp3_ref, ws3_ref, wn3_ref, b3_ref,
          out_ref):
    x = x_ref[0]
    f32 = jnp.float32

    p1 = jax.nn.relu(jnp.dot(x, wp1_ref[...], preferred_element_type=f32)
                     + bp1_ref[...])
    a1 = _shift_down(p1)
    h = (jnp.dot(x, ws1_ref[...], preferred_element_type=f32)
         + jnp.dot(a1, wn1_ref[...], preferred_element_type=f32)
         + b1_ref[...])
    e1 = h * _shift_up(h)

    p3 = jax.nn.relu(jnp.dot(e1, wp3_ref[...], preferred_element_type=f32)
                     + bp3_ref[...])
    a3 = _shift_down(p3)
    h2 = (jnp.dot(e1, ws3_ref[...], preferred_element_type=f32)
          + jnp.dot(a3, wn3_ref[...], preferred_element_type=f32)
          + b3_ref[...])
    e2 = h2 * _shift_up(h2)
    out_ref[0, :, :D] = e2[:L - 2, :]

    d = (dis_ref[0] - DIST_MEAN) / DIST_STD
    loc_d = (pltpu.roll(d, shift=L - 2, axis=0) - d - DIST_MEAN) / DIST_STD
    out_ref[0, :, D:] = loc_d[:L - 2, :]


def kernel(timeid, current_tim, current_dis, loc, attr_t,
           W_pool1, b_pool1, W_self1, W_neigh1, b1,
           W_pool3, b_pool3, W_self3, W_neigh3, b3):
    dis_col = current_dis.reshape(B, L, 1)
    w_spec = pl.BlockSpec((D, D), lambda b: (0, 0))
    bias_spec = pl.BlockSpec((1, D), lambda b: (0, 0))

    return pl.pallas_call(
        _body,
        grid=(B,),
        in_specs=[
            pl.BlockSpec((1, L, 1), lambda b: (b, 0, 0)),
            pl.BlockSpec((1, L, D), lambda b: (b, 0, 0)),
            w_spec, bias_spec, w_spec, w_spec, bias_spec,
            w_spec, bias_spec, w_spec, w_spec, bias_spec,
        ],
        out_specs=pl.BlockSpec((1, L - 2, D + 1), lambda b: (b, 0, 0)),
        out_shape=jax.ShapeDtypeStruct((B, L - 2, D + 1), jnp.float32),
        compiler_params=pltpu.CompilerParams(
            dimension_semantics=("parallel",)),
    )(dis_col, loc,
      W_pool1, b_pool1.reshape(1, D), W_self1, W_neigh1, b1.reshape(1, D),
      W_pool3, b_pool3.reshape(1, D), W_self3, W_neigh3, b3.reshape(1, D))


# speedup vs baseline: 1.2469x; 1.2469x over previous
import jax
import jax.numpy as jnp
from jax.experimental import pallas as pl
from jax.experimental.pallas import tpu as pltpu

B, L, D = 8, 2048, 128
TIME_MEAN, TIME_STD = 43.8756927994, 51.4811932987
DIST_MEAN, DIST_STD = 0.274716042312, 0.127051674693


def _shift_down(a):
    r = pltpu.roll(a, shift=1, axis=0)
    row = jax.lax.broadcasted_iota(jnp.int32, a.shape, 0)
    return jnp.where(row == 0, 0.0, r)


def _shift_up(a):
    r = pltpu.roll(a, shift=a.shape[0] - 1, axis=0)
    row = jax.lax.broadcasted_iota(jnp.int32, a.shape, 0)
    return jnp.where(row == a.shape[0] - 1, 0.0, r)


def _body(dis_ref, x_ref, wp1_ref, bp1_ref, ws1_ref, wn1_ref, b1_ref,
          wp3_ref, bp3_ref, ws3_ref, wn3_ref, b3_ref,
          out_ref):
    x = x_ref[0]
    f32 = jnp.float32

    p1 = jax.nn.relu(jnp.dot(x, wp1_ref[...], preferred_element_type=f32)
                     + bp1_ref[...])
    a1 = _shift_down(p1)
    h = (jnp.dot(x, ws1_ref[...], preferred_element_type=f32)
         + jnp.dot(a1, wn1_ref[...], preferred_element_type=f32)
         + b1_ref[...])
    e1 = h * _shift_up(h)

    p3 = jax.nn.relu(jnp.dot(e1, wp3_ref[...], preferred_element_type=f32)
                     + bp3_ref[...])
    a3 = _shift_down(p3)
    h2 = (jnp.dot(e1, ws3_ref[...], preferred_element_type=f32)
          + jnp.dot(a3, wn3_ref[...], preferred_element_type=f32)
          + b3_ref[...])
    e2 = h2 * _shift_up(h2)
    out_ref[0, :, :D] = e2[:L - 2, :]

    d = (dis_ref[0] - DIST_MEAN) / DIST_STD
    loc_row = (pltpu.roll(d, shift=L - 2, axis=1) - d - DIST_MEAN) / DIST_STD
    out_ref[0, :, D:] = jnp.transpose(loc_row)[:L - 2, :]


def kernel(timeid, current_tim, current_dis, loc, attr_t,
           W_pool1, b_pool1, W_self1, W_neigh1, b1,
           W_pool3, b_pool3, W_self3, W_neigh3, b3):
    dis_row = current_dis.reshape(B, 1, L)
    w_spec = pl.BlockSpec((D, D), lambda b: (0, 0))
    bias_spec = pl.BlockSpec((1, D), lambda b: (0, 0))

    return pl.pallas_call(
        _body,
        grid=(B,),
        in_specs=[
            pl.BlockSpec((1, 1, L), lambda b: (b, 0, 0)),
            pl.BlockSpec((1, L, D), lambda b: (b, 0, 0)),
            w_spec, bias_spec, w_spec, w_spec, bias_spec,
            w_spec, bias_spec, w_spec, w_spec, bias_spec,
        ],
        out_specs=pl.BlockSpec((1, L - 2, D + 1), lambda b: (b, 0, 0)),
        out_shape=jax.ShapeDtypeStruct((B, L - 2, D + 1), jnp.float32),
        compiler_params=pltpu.CompilerParams(
            dimension_semantics=("parallel",)),
    )(dis_row, loc,
      W_pool1, b_pool1.reshape(1, D), W_self1, W_neigh1, b1.reshape(1, D),
      W_pool3, b_pool3.reshape(1, D), W_self3, W_neigh3, b3.reshape(1, D))
